# baseline (speedup 1.0000x reference)
import functools

import jax
import jax.numpy as jnp
from jax import lax
from jax.experimental import pallas as pl
from jax.experimental.pallas import tpu as pltpu
from jax.experimental.pallas import tpu_sc as plsc

N = 10000
E = 320000
D = 128
NSC = 2
NS = 16
ROWS_PER_TILE = N // NS
CHUNK = 96
EDGES_PER_TILE = E // (NSC * NS)
NCHUNK = -(-EDGES_PER_TILE // CHUNK // 2) * 2
E_PAD = NSC * NS * NCHUNK * CHUNK
NROWS = N + 8


def _sc_aggregate(x, src4, dst4):
  mesh = plsc.VectorSubcoreMesh(core_axis_name="c", subcore_axis_name="s")

  @functools.partial(
      pl.kernel,
      mesh=mesh,
      compiler_params=pltpu.CompilerParams(use_tc_tiling_on_sc=False),
      out_type=jax.ShapeDtypeStruct((NSC, N, D), jnp.float32),
      scratch_types=[
          pltpu.VMEM_SHARED((NROWS, D), jnp.float32),
          pltpu.VMEM((NCHUNK, CHUNK), jnp.int32),
          pltpu.VMEM((NCHUNK, CHUNK), jnp.int32),
          pltpu.VMEM((CHUNK, D), jnp.float32),
          pltpu.VMEM((CHUNK, D), jnp.float32),
          pltpu.SemaphoreType.DMA,
          pltpu.SemaphoreType.DMA,
          pltpu.SemaphoreType.DMA,
          pltpu.SemaphoreType.DMA,
      ],
  )
  def k(x_hbm, src_hbm, dst_hbm, h2_hbm, agg_s, src_v, dst_v,
        rows0, rows1, gsem0, gsem1, ssem0, ssem1):
    c = lax.axis_index("c")
    s = lax.axis_index("s")
    r0 = s * ROWS_PER_TILE
    pltpu.sync_copy(x_hbm.at[pl.ds(r0, ROWS_PER_TILE)],
                    agg_s.at[pl.ds(r0, ROWS_PER_TILE)])
    pltpu.sync_copy(src_hbm.at[c, s], src_v)
    pltpu.sync_copy(dst_hbm.at[c, s], dst_v)
    plsc.subcore_barrier()

    bufs = (rows0, rows1)
    gsems = (gsem0, gsem1)
    ssems = (ssem0, ssem1)

    def gather(j, b):
      pltpu.async_copy(x_hbm.at[src_v.at[j]], bufs[b], gsems[b])

    def wait_gather(b):
      pltpu.make_async_copy(x_hbm.at[src_v.at[0]], bufs[b], gsems[b]).wait()

    def scatter(j, b):
      pltpu.async_copy(bufs[b], agg_s.at[dst_v.at[j]], ssems[b], add=True)

    def wait_scatter(b):
      pltpu.make_async_copy(bufs[b], agg_s.at[dst_v.at[0]], ssems[b]).wait()

    gather(0, 0)
    gather(1, 1)

    def step(jj, carry):
      for b in range(2):
        j = 2 * jj + b
        wait_gather(b)
        scatter(j, b)
        wait_scatter(b)
        gather(j + 2, b)
      return carry

    lax.fori_loop(0, NCHUNK // 2 - 1, step, 0)
    for j in (NCHUNK - 2, NCHUNK - 1):
      b = j % 2
      wait_gather(b)
      scatter(j, b)
      wait_scatter(b)

    plsc.subcore_barrier()
    pltpu.sync_copy(agg_s.at[pl.ds(r0, ROWS_PER_TILE)],
                    h2_hbm.at[c, pl.ds(r0, ROWS_PER_TILE)])

  return k(x, src4, dst4)


def _mlp_body(ha_ref, hb_ref, x_ref, w1_ref, b1_ref, w2_ref, b2_ref, o_ref):
  h = ha_ref[0] + hb_ref[0] - x_ref[...]
  a = jnp.dot(h, w1_ref[...], preferred_element_type=jnp.float32) + b1_ref[...]
  a = jnp.maximum(a, 0.0)
  o_ref[...] = jnp.dot(a, w2_ref[...], preferred_element_type=jnp.float32) + b2_ref[...]


def _mlp(h2, x, W1, b1, W2, b2):
  blk = 1000
  return pl.pallas_call(
      _mlp_body,
      grid=(N // blk,),
      in_specs=[
          pl.BlockSpec((1, blk, D), lambda i: (0, i, 0)),
          pl.BlockSpec((1, blk, D), lambda i: (1, i, 0)),
          pl.BlockSpec((blk, D), lambda i: (i, 0)),
          pl.BlockSpec((D, D), lambda i: (0, 0)),
          pl.BlockSpec((1, D), lambda i: (0, 0)),
          pl.BlockSpec((D, D), lambda i: (0, 0)),
          pl.BlockSpec((1, D), lambda i: (0, 0)),
      ],
      out_specs=pl.BlockSpec((blk, D), lambda i: (i, 0)),
      out_shape=jax.ShapeDtypeStruct((N, D), jnp.float32),
  )(h2, h2, x, W1, b1, W2, b2)


def kernel(x, edge_index, W1, b1, W2, b2):
  npad = E_PAD - E
  src = jnp.concatenate([edge_index[0], jnp.zeros((npad,), jnp.int32)])
  dst = jnp.concatenate([edge_index[1], jnp.full((npad,), N, jnp.int32)])
  src4 = src.reshape(NSC, NS, NCHUNK, CHUNK)
  dst4 = dst.reshape(NSC, NS, NCHUNK, CHUNK)
  h2 = _sc_aggregate(x, src4, dst4)
  return _mlp(h2, x, W1, b1.reshape(1, D), W2, b2.reshape(1, D))


# speedup vs baseline: 1.0106x; 1.0106x over previous
import functools

import jax
import jax.numpy as jnp
from jax import lax
from jax.experimental import pallas as pl
from jax.experimental.pallas import tpu as pltpu
from jax.experimental.pallas import tpu_sc as plsc

N = 10000
E = 320000
D = 128
NSC = 2
NS = 16
ROWS_PER_TILE = N // NS
CHUNK = 96
EDGES_PER_TILE = E // (NSC * NS)
NCHUNK = -(-EDGES_PER_TILE // CHUNK // 2) * 2
E_PAD = NSC * NS * NCHUNK * CHUNK
NROWS = N + 8


def _sc_aggregate(x, src4, dst4):
  mesh = plsc.VectorSubcoreMesh(core_axis_name="c", subcore_axis_name="s")

  @functools.partial(
      pl.kernel,
      mesh=mesh,
      compiler_params=pltpu.CompilerParams(use_tc_tiling_on_sc=False),
      out_type=jax.ShapeDtypeStruct((NSC, N, D), jnp.float32),
      scratch_types=[
          pltpu.VMEM_SHARED((NROWS, D), jnp.float32),
          pltpu.VMEM((NCHUNK, CHUNK), jnp.int32),
          pltpu.VMEM((NCHUNK, CHUNK), jnp.int32),
          pltpu.VMEM((CHUNK, D), jnp.float32),
          pltpu.VMEM((CHUNK, D), jnp.float32),
          pltpu.SemaphoreType.DMA,
          pltpu.SemaphoreType.DMA,
          pltpu.SemaphoreType.DMA,
          pltpu.SemaphoreType.DMA,
      ],
  )
  def k(x_hbm, src_hbm, dst_hbm, h2_hbm, agg_s, src_v, dst_v,
        rows0, rows1, gsem0, gsem1, ssem0, ssem1):
    c = lax.axis_index("c")
    s = lax.axis_index("s")
    r0 = s * ROWS_PER_TILE
    pltpu.sync_copy(x_hbm.at[pl.ds(r0, ROWS_PER_TILE)],
                    agg_s.at[pl.ds(r0, ROWS_PER_TILE)])
    pltpu.sync_copy(src_hbm.at[c, s], src_v)
    pltpu.sync_copy(dst_hbm.at[c, s], dst_v)
    plsc.subcore_barrier()

    bufs = (rows0, rows1)
    gsems = (gsem0, gsem1)
    ssems = (ssem0, ssem1)

    def gather(j, b):
      pltpu.async_copy(x_hbm.at[src_v.at[j]], bufs[b], gsems[b])

    def wait_gather(b):
      pltpu.make_async_copy(x_hbm.at[src_v.at[0]], bufs[b], gsems[b]).wait()

    def scatter(j, b):
      pltpu.async_copy(bufs[b], agg_s.at[dst_v.at[j]], ssems[b], add=True)

    def wait_scatter(b):
      pltpu.make_async_copy(bufs[b], agg_s.at[dst_v.at[0]], ssems[b]).wait()

    gather(0, 0)
    gather(1, 1)

    def step(jj, carry):
      for b in range(2):
        j = 2 * jj + b
        wait_gather(b)
        gather(j + 2, b)
      return carry

    lax.fori_loop(0, NCHUNK // 2 - 1, step, 0)
    for j in (NCHUNK - 2, NCHUNK - 1):
      b = j % 2
      wait_gather(b)

    plsc.subcore_barrier()
    pltpu.sync_copy(agg_s.at[pl.ds(r0, ROWS_PER_TILE)],
                    h2_hbm.at[c, pl.ds(r0, ROWS_PER_TILE)])

  return k(x, src4, dst4)


def _mlp_body(ha_ref, hb_ref, x_ref, w1_ref, b1_ref, w2_ref, b2_ref, o_ref):
  h = ha_ref[0] + hb_ref[0] - x_ref[...]
  a = jnp.dot(h, w1_ref[...], preferred_element_type=jnp.float32) + b1_ref[...]
  a = jnp.maximum(a, 0.0)
  o_ref[...] = jnp.dot(a, w2_ref[...], preferred_element_type=jnp.float32) + b2_ref[...]


def _mlp(h2, x, W1, b1, W2, b2):
  blk = 1000
  return pl.pallas_call(
      _mlp_body,
      grid=(N // blk,),
      in_specs=[
          pl.BlockSpec((1, blk, D), lambda i: (0, i, 0)),
          pl.BlockSpec((1, blk, D), lambda i: (1, i, 0)),
          pl.BlockSpec((blk, D), lambda i: (i, 0)),
          pl.BlockSpec((D, D), lambda i: (0, 0)),
          pl.BlockSpec((1, D), lambda i: (0, 0)),
          pl.BlockSpec((D, D), lambda i: (0, 0)),
          pl.BlockSpec((1, D), lambda i: (0, 0)),
      ],
      out_specs=pl.BlockSpec((blk, D), lambda i: (i, 0)),
      out_shape=jax.ShapeDtypeStruct((N, D), jnp.float32),
  )(h2, h2, x, W1, b1, W2, b2)


def kernel(x, edge_index, W1, b1, W2, b2):
  npad = E_PAD - E
  src = jnp.concatenate([edge_index[0], jnp.zeros((npad,), jnp.int32)])
  dst = jnp.concatenate([edge_index[1], jnp.full((npad,), N, jnp.int32)])
  src4 = src.reshape(NSC, NS, NCHUNK, CHUNK)
  dst4 = dst.reshape(NSC, NS, NCHUNK, CHUNK)
  h2 = _sc_aggregate(x, src4, dst4)
  return _mlp(h2, x, W1, b1.reshape(1, D), W2, b2.reshape(1, D))


# speedup vs baseline: 1.5106x; 1.4948x over previous
import functools

import jax
import jax.numpy as jnp
from jax import lax
from jax.experimental import pallas as pl
from jax.experimental.pallas import tpu as pltpu
from jax.experimental.pallas import tpu_sc as plsc

N = 10000
E = 320000
D = 128
COLS = D // 2
NS = 16
ROWS_PER_TILE = N // NS
CHUNK = 128
PERIOD = 8
HBM_K = 3
NCHUNK = 160
E_PAD = NS * NCHUNK * CHUNK
NROWS = N + 8


def _sc_aggregate(x2, sd4):
  mesh = plsc.VectorSubcoreMesh(core_axis_name="c", subcore_axis_name="s")

  @functools.partial(
      pl.kernel,
      mesh=mesh,
      compiler_params=pltpu.CompilerParams(use_tc_tiling_on_sc=False),
      out_type=jax.ShapeDtypeStruct((N, D), jnp.float32),
      scratch_types=[
          pltpu.VMEM_SHARED((N, COLS), jnp.float32),
          pltpu.VMEM_SHARED((NROWS, COLS), jnp.float32),
          pltpu.VMEM((2, CHUNK), jnp.int32),
          pltpu.VMEM((2, CHUNK), jnp.int32),
          pltpu.VMEM((2, CHUNK), jnp.int32),
          pltpu.VMEM((2, CHUNK), jnp.int32),
          pltpu.VMEM((CHUNK, COLS), jnp.float32),
          pltpu.VMEM((CHUNK, COLS), jnp.float32),
          pltpu.SemaphoreType.DMA,
          pltpu.SemaphoreType.DMA,
          pltpu.SemaphoreType.DMA,
          pltpu.SemaphoreType.DMA,
          pltpu.SemaphoreType.DMA,
          pltpu.SemaphoreType.DMA,
          pltpu.SemaphoreType.DMA,
          pltpu.SemaphoreType.DMA,
      ],
  )
  def k(x2_hbm, sd_hbm, h_hbm, x_s, agg_s, i0, i1, i2, i3,
        rows0, rows1, gsem0, gsem1, ssem0, ssem1, is0, is1, is2, is3):
    c = lax.axis_index("c")
    s = lax.axis_index("s")
    r0 = s * ROWS_PER_TILE
    c0 = c * COLS
    x_hbm = x2_hbm.at[c]
    pltpu.sync_copy(x_hbm.at[pl.ds(r0, ROWS_PER_TILE)],
                    x_s.at[pl.ds(r0, ROWS_PER_TILE)])
    pltpu.sync_copy(x_hbm.at[pl.ds(r0, ROWS_PER_TILE)],
                    agg_s.at[pl.ds(r0, ROWS_PER_TILE)])
    plsc.subcore_barrier()

    ibufs = (i0, i1, i2, i3)
    isems = (is0, is1, is2, is3)
    bufs = (rows0, rows1)
    gsems = (gsem0, gsem1)
    ssems = (ssem0, ssem1)

    def iload(j):
      sl = j % 4
      pltpu.async_copy(sd_hbm.at[s, j], ibufs[sl], isems[sl])

    def wait_iload(j):
      sl = j % 4
      pltpu.make_async_copy(sd_hbm.at[s, 0], ibufs[sl], isems[sl]).wait()

    def from_hbm(j):
      return (j % PERIOD) < HBM_K

    def gather(j, b):
      table = x_hbm if from_hbm(j) else x_s
      pltpu.async_copy(table.at[ibufs[j % 4].at[0]], bufs[b], gsems[b])

    def wait_gather(j, b):
      table = x_hbm if from_hbm(j) else x_s
      pltpu.make_async_copy(table.at[ibufs[j % 4].at[0]], bufs[b],
                            gsems[b]).wait()

    def scatter(j, b):
      pltpu.async_copy(bufs[b], agg_s.at[ibufs[j % 4].at[1]], ssems[b],
                       add=True)

    def wait_scatter(b):
      pltpu.make_async_copy(bufs[b], agg_s.at[ibufs[0].at[1]],
                            ssems[b]).wait()

    for j in range(4):
      iload(j)
    wait_iload(0)
    gather(0, 0)
    wait_iload(1)
    gather(1, 1)

    def step(jj, carry):
      for bb in range(PERIOD):
        j = PERIOD * jj + bb
        b = bb % 2
        wait_gather(bb, b)
        scatter(bb, b)
        wait_scatter(b)
        iload_j = PERIOD * jj + bb + 4
        sl_src = sd_hbm.at[s, iload_j]
        pltpu.async_copy(sl_src, ibufs[bb % 4], isems[bb % 4])
        wait_iload(bb + 2)
        gather(bb + 2, b)
      return carry

    lax.fori_loop(0, NCHUNK // PERIOD - 1, step, 0)
    for j in range(NCHUNK - PERIOD, NCHUNK):
      b = j % 2
      wait_gather(j, b)
      scatter(j, b)
      wait_scatter(b)
      if j + 4 < NCHUNK:
        iload(j + 4)
      if j + 2 < NCHUNK:
        wait_iload(j + 2)
        gather(j + 2, b)

    plsc.subcore_barrier()
    pltpu.sync_copy(agg_s.at[pl.ds(r0, ROWS_PER_TILE)],
                    h_hbm.at[pl.ds(r0, ROWS_PER_TILE), pl.ds(c0, COLS)])

  return k(x2, sd4)


def _mlp_body(h_ref, w1_ref, b1_ref, w2_ref, b2_ref, o_ref):
  h = h_ref[...]
  a = jnp.dot(h, w1_ref[...], preferred_element_type=jnp.float32) + b1_ref[...]
  a = jnp.maximum(a, 0.0)
  o_ref[...] = jnp.dot(a, w2_ref[...], preferred_element_type=jnp.float32) + b2_ref[...]


def _mlp(h, W1, b1, W2, b2):
  blk = 1000
  return pl.pallas_call(
      _mlp_body,
      grid=(N // blk,),
      in_specs=[
          pl.BlockSpec((blk, D), lambda i: (i, 0)),
          pl.BlockSpec((D, D), lambda i: (0, 0)),
          pl.BlockSpec((1, D), lambda i: (0, 0)),
          pl.BlockSpec((D, D), lambda i: (0, 0)),
          pl.BlockSpec((1, D), lambda i: (0, 0)),
      ],
      out_specs=pl.BlockSpec((blk, D), lambda i: (i, 0)),
      out_shape=jax.ShapeDtypeStruct((N, D), jnp.float32),
  )(h, W1, b1, W2, b2)


def kernel(x, edge_index, W1, b1, W2, b2):
  npad = E_PAD - E
  src = jnp.concatenate([edge_index[0], jnp.zeros((npad,), jnp.int32)])
  dst = jnp.concatenate([edge_index[1], jnp.full((npad,), N, jnp.int32)])
  sd4 = jnp.stack([src.reshape(NS, NCHUNK, CHUNK),
                   dst.reshape(NS, NCHUNK, CHUNK)], axis=2)
  x2 = jnp.stack([x[:, :COLS], x[:, COLS:]])
  h = _sc_aggregate(x2, sd4)
  return _mlp(h, W1, b1.reshape(1, D), W2, b2.reshape(1, D))
